# Initial kernel scaffold; baseline (speedup 1.0000x reference)
import functools

import jax
import jax.numpy as jnp
from jax import lax
from jax.experimental import pallas as pl
from jax.experimental.pallas import tpu as pltpu
from jax.experimental.pallas import tpu_sc as plsc

_VOCAB = 1000000
_DIM = 32
_MAXLEN = 200
_B = 4096
_NCLS = 10
_BN_EPS = 1e-5

_NC = 2
_NS = 16
_NW = _NC * _NS

_ROWS_PER_W = _B // _NW
_WIN = 100
_WPR = _MAXLEN // _WIN
_CHUNK_B = 4
_CHUNK_W = _CHUNK_B * _WPR
_N_ITERS = _ROWS_PER_W // _CHUNK_B


def _sc_gather_sum(idx2d, table):
    n_pos = _B * _MAXLEN
    mesh = plsc.VectorSubcoreMesh(core_axis_name="c", subcore_axis_name="s")

    @functools.partial(
        pl.kernel,
        out_type=(
            jax.ShapeDtypeStruct((n_pos, _DIM), jnp.float32),
            jax.ShapeDtypeStruct((_B, _DIM), jnp.float32),
        ),
        mesh=mesh,
        scratch_types=[
            pltpu.VMEM((_CHUNK_W, _WIN), jnp.int32),
            pltpu.VMEM((_CHUNK_B * _MAXLEN, _DIM), jnp.float32),
            pltpu.VMEM((_ROWS_PER_W, _DIM), jnp.float32),
            pltpu.SemaphoreType.DMA,
        ],
    )
    def sc_kernel(idx_hbm, tab_hbm, raw_hbm, sums_hbm, idx_v, rows_v, acc_v, gsem):
        wid = lax.axis_index("s") * _NC + lax.axis_index("c")
        w_row0 = wid * (_ROWS_PER_W * _WPR)

        @pl.loop(0, _N_ITERS)
        def _(c):
            r0 = w_row0 + c * _CHUNK_W
            pltpu.sync_copy(idx_hbm.at[pl.ds(r0, _CHUNK_W)], idx_v)
            copies = []
            for j in range(_CHUNK_W):
                copies.append(
                    pltpu.async_copy(
                        tab_hbm.at[idx_v.at[j]],
                        rows_v.at[pl.ds(j * _WIN, _WIN)],
                        gsem,
                    )
                )
            for cp in copies:
                cp.wait()
            pltpu.sync_copy(rows_v, raw_hbm.at[pl.ds(r0 * _WIN, _CHUNK_B * _MAXLEN)])
            zero = jnp.zeros((16,), jnp.float32)
            for b in range(_CHUNK_B):
                base = b * _MAXLEN

                def body(p, accs, base=base):
                    s00, s01, s10, s11 = accs
                    q = base + p * 4
                    s00 = s00 + rows_v[q, pl.ds(0, 16)]
                    s01 = s01 + rows_v[q, pl.ds(16, 16)]
                    s10 = s10 + rows_v[q + 1, pl.ds(0, 16)]
                    s11 = s11 + rows_v[q + 1, pl.ds(16, 16)]
                    s00 = s00 + rows_v[q + 2, pl.ds(0, 16)]
                    s01 = s01 + rows_v[q + 2, pl.ds(16, 16)]
                    s10 = s10 + rows_v[q + 3, pl.ds(0, 16)]
                    s11 = s11 + rows_v[q + 3, pl.ds(16, 16)]
                    return s00, s01, s10, s11

                s00, s01, s10, s11 = lax.fori_loop(
                    0, _MAXLEN // 4, body, (zero, zero, zero, zero)
                )
                row = c * _CHUNK_B + b
                acc_v[row, pl.ds(0, 16)] = s00 + s10
                acc_v[row, pl.ds(16, 16)] = s01 + s11

        pltpu.sync_copy(acc_v, sums_hbm.at[pl.ds(wid * _ROWS_PER_W, _ROWS_PER_W)])

    return sc_kernel(idx2d, table)


def _tc_head_body(sums_ref, gamma_ref, beta_ref, mean_ref, var_ref, fcw_ref,
                  fcb_ref, xavg_ref, bn_ref, final_ref):
    x_avg = sums_ref[...] * (1.0 / _MAXLEN)
    xavg_ref[...] = x_avg
    bn = (x_avg - mean_ref[...]) / jnp.sqrt(var_ref[...] + _BN_EPS) \
        * gamma_ref[...] + beta_ref[...]
    bn_ref[...] = bn
    final_ref[...] = lax.dot_general(
        bn, fcw_ref[...],
        dimension_numbers=(((1,), (1,)), ((), ())),
        preferred_element_type=jnp.float32,
    ) + fcb_ref[...]


def _tc_head(sums, bn_gamma, bn_beta, bn_mean, bn_var, fc_w, fc_b):
    f32 = jnp.float32
    return pl.pallas_call(
        _tc_head_body,
        out_shape=[
            jax.ShapeDtypeStruct((_B, _DIM), f32),
            jax.ShapeDtypeStruct((_B, _DIM), f32),
            jax.ShapeDtypeStruct((_B, _NCLS), f32),
        ],
    )(
        sums,
        bn_gamma.reshape(1, _DIM),
        bn_beta.reshape(1, _DIM),
        bn_mean.reshape(1, _DIM),
        bn_var.reshape(1, _DIM),
        fc_w,
        fc_b.reshape(1, _NCLS),
    )


def kernel(word_idx, table, bn_gamma, bn_beta, bn_mean, bn_var, fc_w, fc_b):
    idx2d = word_idx.reshape(_B * _MAXLEN // _WIN, _WIN)
    raw_flat, sums = _sc_gather_sum(idx2d, table)
    raw_feature = raw_flat.reshape(_B, _MAXLEN, _DIM)
    x_avg, x_avg_bn, x_final = _tc_head(
        sums, bn_gamma, bn_beta, bn_mean, bn_var, fc_w, fc_b
    )
    return (x_final, x_avg_bn, x_avg, raw_feature)


# speedup vs baseline: 1.4315x; 1.4315x over previous
import functools

import jax
import jax.numpy as jnp
from jax import lax
from jax.experimental import pallas as pl
from jax.experimental.pallas import tpu as pltpu
from jax.experimental.pallas import tpu_sc as plsc

_VOCAB = 1000000
_DIM = 32
_MAXLEN = 200
_B = 4096
_NCLS = 10
_BN_EPS = 1e-5

_NC = 2
_NS = 16
_NW = _NC * _NS

_ROWS_PER_W = _B // _NW
_WIN = 100
_WPR = _MAXLEN // _WIN
_CHUNK_B = 4
_CHUNK_W = _CHUNK_B * _WPR
_N_ITERS = _ROWS_PER_W // _CHUNK_B


def _sc_gather_sum(idx2d, table):
    n_pos = _B * _MAXLEN
    mesh = plsc.VectorSubcoreMesh(core_axis_name="c", subcore_axis_name="s")

    @functools.partial(
        pl.kernel,
        out_type=(
            jax.ShapeDtypeStruct((n_pos, _DIM), jnp.float32),
            jax.ShapeDtypeStruct((_B, _DIM), jnp.float32),
        ),
        mesh=mesh,
        scratch_types=[
            pltpu.VMEM((_CHUNK_W, _WIN), jnp.int32),
            pltpu.VMEM((_CHUNK_B * _MAXLEN, _DIM), jnp.float32),
            pltpu.VMEM((_ROWS_PER_W, _DIM), jnp.float32),
            pltpu.SemaphoreType.DMA,
        ],
        compiler_params=pltpu.CompilerParams(use_tc_tiling_on_sc=False),
    )
    def sc_kernel(idx_hbm, tab_hbm, raw_hbm, sums_hbm, idx_v, rows_v, acc_v, gsem):
        wid = lax.axis_index("s") * _NC + lax.axis_index("c")
        w_row0 = wid * (_ROWS_PER_W * _WPR)

        @pl.loop(0, _N_ITERS)
        def _(c):
            r0 = w_row0 + c * _CHUNK_W
            pltpu.sync_copy(idx_hbm.at[pl.ds(r0, _CHUNK_W)], idx_v)
            copies = []
            for j in range(_CHUNK_W):
                copies.append(
                    pltpu.async_copy(
                        tab_hbm.at[idx_v.at[j]],
                        rows_v.at[pl.ds(j * _WIN, _WIN)],
                        gsem,
                    )
                )
            for cp in copies:
                cp.wait()
            pltpu.sync_copy(rows_v, raw_hbm.at[pl.ds(r0 * _WIN, _CHUNK_B * _MAXLEN)])
            zero = jnp.zeros((16,), jnp.float32)
            for b in range(_CHUNK_B):
                base = b * _MAXLEN

                def body(p, accs, base=base):
                    s00, s01, s10, s11 = accs
                    q = base + p * 4
                    s00 = s00 + rows_v[q, pl.ds(0, 16)]
                    s01 = s01 + rows_v[q, pl.ds(16, 16)]
                    s10 = s10 + rows_v[q + 1, pl.ds(0, 16)]
                    s11 = s11 + rows_v[q + 1, pl.ds(16, 16)]
                    s00 = s00 + rows_v[q + 2, pl.ds(0, 16)]
                    s01 = s01 + rows_v[q + 2, pl.ds(16, 16)]
                    s10 = s10 + rows_v[q + 3, pl.ds(0, 16)]
                    s11 = s11 + rows_v[q + 3, pl.ds(16, 16)]
                    return s00, s01, s10, s11

                s00, s01, s10, s11 = lax.fori_loop(
                    0, _MAXLEN // 4, body, (zero, zero, zero, zero)
                )
                row = c * _CHUNK_B + b
                acc_v[row, pl.ds(0, 16)] = s00 + s10
                acc_v[row, pl.ds(16, 16)] = s01 + s11

        pltpu.sync_copy(acc_v, sums_hbm.at[pl.ds(wid * _ROWS_PER_W, _ROWS_PER_W)])

    return sc_kernel(idx2d, table)


def _tc_head_body(sums_ref, gamma_ref, beta_ref, mean_ref, var_ref, fcw_ref,
                  fcb_ref, xavg_ref, bn_ref, final_ref):
    x_avg = sums_ref[...] * (1.0 / _MAXLEN)
    xavg_ref[...] = x_avg
    bn = (x_avg - mean_ref[...]) / jnp.sqrt(var_ref[...] + _BN_EPS) \
        * gamma_ref[...] + beta_ref[...]
    bn_ref[...] = bn
    final_ref[...] = lax.dot_general(
        bn, fcw_ref[...],
        dimension_numbers=(((1,), (1,)), ((), ())),
        preferred_element_type=jnp.float32,
    ) + fcb_ref[...]


def _tc_head(sums, bn_gamma, bn_beta, bn_mean, bn_var, fc_w, fc_b):
    f32 = jnp.float32
    return pl.pallas_call(
        _tc_head_body,
        out_shape=[
            jax.ShapeDtypeStruct((_B, _DIM), f32),
            jax.ShapeDtypeStruct((_B, _DIM), f32),
            jax.ShapeDtypeStruct((_B, _NCLS), f32),
        ],
    )(
        sums,
        bn_gamma.reshape(1, _DIM),
        bn_beta.reshape(1, _DIM),
        bn_mean.reshape(1, _DIM),
        bn_var.reshape(1, _DIM),
        fc_w,
        fc_b.reshape(1, _NCLS),
    )


def kernel(word_idx, table, bn_gamma, bn_beta, bn_mean, bn_var, fc_w, fc_b):
    idx2d = word_idx.reshape(_B * _MAXLEN // _WIN, _WIN)
    raw_flat, sums = _sc_gather_sum(idx2d, table)
    raw_feature = raw_flat.reshape(_B, _MAXLEN, _DIM)
    x_avg, x_avg_bn, x_final = _tc_head(
        sums, bn_gamma, bn_beta, bn_mean, bn_var, fc_w, fc_b
    )
    return (x_final, x_avg_bn, x_avg, raw_feature)


# speedup vs baseline: 1.4573x; 1.0180x over previous
import functools

import jax
import jax.numpy as jnp
from jax import lax
from jax.experimental import pallas as pl
from jax.experimental.pallas import tpu as pltpu
from jax.experimental.pallas import tpu_sc as plsc

_VOCAB = 1000000
_DIM = 32
_MAXLEN = 200
_B = 4096
_NCLS = 10
_BN_EPS = 1e-5

_NC = 2
_NS = 16
_NW = _NC * _NS
_BB = _B // _NW


def _sc_gather_transpose(idx3d, table):
    mesh = plsc.VectorSubcoreMesh(core_axis_name="c", subcore_axis_name="s")

    @functools.partial(
        pl.kernel,
        out_type=jax.ShapeDtypeStruct((_MAXLEN, 4, _NW, 1024), jnp.float32),
        mesh=mesh,
        scratch_types=[
            pltpu.VMEM((_MAXLEN, _BB), jnp.int32),
            pltpu.VMEM((_BB, _DIM), jnp.float32),
            pltpu.VMEM((_BB, _DIM), jnp.float32),
            pltpu.VMEM((4 * 1024,), jnp.float32),
            pltpu.VMEM((4 * 1024,), jnp.float32),
            pltpu.SemaphoreType.DMA,
            pltpu.SemaphoreType.DMA,
            pltpu.SemaphoreType.DMA,
            pltpu.SemaphoreType.DMA,
        ],
        compiler_params=pltpu.CompilerParams(
            use_tc_tiling_on_sc=False, needs_layout_passes=False
        ),
    )
    def sc_kernel(idx_hbm, tab_hbm, out_hbm, idx_v, rows0, rows1, tiles0,
                  tiles1, sg0, sg1, sw0, sw1):
        w = lax.axis_index("s") * _NC + lax.axis_index("c")
        pltpu.sync_copy(idx_hbm.at[w], idx_v)

        i16 = lax.iota(jnp.int32, 16)
        sidx0 = (i16 // 8) * 1024 + (i16 % 8) * 128
        sidx1 = sidx0 + 2048

        rows = (rows0, rows1)
        tiles = (tiles0, tiles1)
        sg = (sg0, sg1)
        sw = (sw0, sw1)

        def fire_gather(l, par):
            pltpu.async_copy(tab_hbm.at[idx_v.at[l]], rows[par], sg[par])

        def wait_gather(l, par):
            pltpu.make_async_copy(tab_hbm.at[idx_v.at[l]], rows[par],
                                  sg[par]).wait()

        def fire_writes(l, par):
            for k in range(4):
                pltpu.async_copy(tiles[par].at[pl.ds(k * 1024, 1024)],
                                 out_hbm.at[l, k, w], sw[par])

        def wait_writes(l, par):
            for k in range(4):
                pltpu.make_async_copy(tiles[par].at[pl.ds(k * 1024, 1024)],
                                      out_hbm.at[l, k, w], sw[par]).wait()

        fire_gather(0, 0)
        fire_gather(1, 1)

        @pl.loop(0, _MAXLEN // 2)
        def _(i):
            for par in range(2):
                l = 2 * i + par

                @pl.when(i >= 1)
                def _():
                    wait_writes(l - 2, par)

                wait_gather(l, par)

                @pl.loop(0, _BB, step=8)
                def _(p0):
                    for t in range(8):
                        p = p0 + t
                        r0 = rows[par][p, pl.ds(0, 16)]
                        r1 = rows[par][p, pl.ds(16, 16)]
                        plsc.store_scatter(tiles[par], [sidx0 + p], r0)
                        plsc.store_scatter(tiles[par], [sidx1 + p], r1)

                @pl.when(i < _MAXLEN // 2 - 1)
                def _():
                    fire_gather(l + 2, par)

                fire_writes(l, par)

        wait_writes(_MAXLEN - 2, 0)
        wait_writes(_MAXLEN - 1, 1)

    return sc_kernel(idx3d, table)


def _tc_reduce_body(raw_ref, avg_ref):
    i = pl.program_id(0)

    @pl.when(i == 0)
    def _():
        avg_ref[...] = jnp.zeros_like(avg_ref)

    avg_ref[...] += jnp.sum(raw_ref[...], axis=0)

    @pl.when(i == pl.num_programs(0) - 1)
    def _():
        avg_ref[...] *= 1.0 / _MAXLEN


def _tc_reduce(raw5):
    lblk = 8
    return pl.pallas_call(
        _tc_reduce_body,
        grid=(_MAXLEN // lblk,),
        in_specs=[
            pl.BlockSpec((lblk, 4, _NW, 8, 128), lambda i: (i, 0, 0, 0, 0))
        ],
        out_specs=pl.BlockSpec((4, _NW, 8, 128), lambda i: (0, 0, 0, 0)),
        out_shape=jax.ShapeDtypeStruct((4, _NW, 8, 128), jnp.float32),
    )(raw5)


def _tc_head_body(xavg_ref, gamma_ref, beta_ref, mean_ref, var_ref, fcw_ref,
                  fcb_ref, bn_ref, final_ref):
    x_avg = xavg_ref[...]
    bn = (x_avg - mean_ref[...]) / jnp.sqrt(var_ref[...] + _BN_EPS) \
        * gamma_ref[...] + beta_ref[...]
    bn_ref[...] = bn
    final_ref[...] = lax.dot_general(
        bn, fcw_ref[...],
        dimension_numbers=(((1,), (1,)), ((), ())),
        preferred_element_type=jnp.float32,
    ) + fcb_ref[...]


def _tc_head(x_avg, bn_gamma, bn_beta, bn_mean, bn_var, fc_w, fc_b):
    f32 = jnp.float32
    return pl.pallas_call(
        _tc_head_body,
        out_shape=[
            jax.ShapeDtypeStruct((_B, _DIM), f32),
            jax.ShapeDtypeStruct((_B, _NCLS), f32),
        ],
    )(
        x_avg,
        bn_gamma.reshape(1, _DIM),
        bn_beta.reshape(1, _DIM),
        bn_mean.reshape(1, _DIM),
        bn_var.reshape(1, _DIM),
        fc_w,
        fc_b.reshape(1, _NCLS),
    )


def kernel(word_idx, table, bn_gamma, bn_beta, bn_mean, bn_var, fc_w, fc_b):
    idx3d = word_idx.reshape(_NW, _BB, _MAXLEN).transpose(0, 2, 1)
    out = _sc_gather_transpose(idx3d, table)
    raw5 = out.reshape(_MAXLEN, 4, _NW, 8, 128)
    raw_feature = raw5.transpose(2, 4, 0, 1, 3).reshape(_B, _MAXLEN, _DIM)
    avg4 = _tc_reduce(raw5)
    x_avg = avg4.transpose(1, 3, 0, 2).reshape(_B, _DIM)
    x_avg_bn, x_final = _tc_head(
        x_avg, bn_gamma, bn_beta, bn_mean, bn_var, fc_w, fc_b
    )
    return (x_final, x_avg_bn, x_avg, raw_feature)


# speedup vs baseline: 1.4586x; 1.0009x over previous
import functools

import jax
import jax.numpy as jnp
from jax import lax
from jax.experimental import pallas as pl
from jax.experimental.pallas import tpu as pltpu
from jax.experimental.pallas import tpu_sc as plsc

_VOCAB = 1000000
_DIM = 32
_MAXLEN = 200
_B = 4096
_NCLS = 10
_BN_EPS = 1e-5

_NC = 2
_NS = 16
_NW = _NC * _NS
_BB = _B // _NW


def _sc_gather_transpose(idx3d, table):
    mesh = plsc.VectorSubcoreMesh(core_axis_name="c", subcore_axis_name="s")

    @functools.partial(
        pl.kernel,
        out_type=jax.ShapeDtypeStruct((_MAXLEN, 4, _NW, 1024), jnp.float32),
        mesh=mesh,
        scratch_types=[
            pltpu.VMEM((_MAXLEN, _BB), jnp.int32),
        ]
        + [pltpu.VMEM((_BB, _DIM), jnp.float32) for _ in range(4)]
        + [pltpu.VMEM((4 * 1024,), jnp.float32) for _ in range(4)]
        + [pltpu.SemaphoreType.DMA for _ in range(8)],
        compiler_params=pltpu.CompilerParams(
            use_tc_tiling_on_sc=False, needs_layout_passes=False
        ),
    )
    def sc_kernel(idx_hbm, tab_hbm, out_hbm, idx_v, rows0, rows1, rows2,
                  rows3, tiles0, tiles1, tiles2, tiles3, sg0, sg1, sg2, sg3,
                  sw0, sw1, sw2, sw3):
        w = lax.axis_index("s") * _NC + lax.axis_index("c")
        pltpu.sync_copy(idx_hbm.at[w], idx_v)

        i16 = lax.iota(jnp.int32, 16)
        sidx0 = (i16 // 8) * 1024 + (i16 % 8) * 128
        sidx1 = sidx0 + 2048

        rows = (rows0, rows1, rows2, rows3)
        tiles = (tiles0, tiles1, tiles2, tiles3)
        sg = (sg0, sg1, sg2, sg3)
        sw = (sw0, sw1, sw2, sw3)

        def fire_gather(l, j):
            pltpu.async_copy(tab_hbm.at[idx_v.at[l]], rows[j], sg[j])

        def wait_gather(l, j):
            pltpu.make_async_copy(tab_hbm.at[idx_v.at[l]], rows[j],
                                  sg[j]).wait()

        def fire_writes(l, j):
            for k in range(4):
                pltpu.async_copy(tiles[j].at[pl.ds(k * 1024, 1024)],
                                 out_hbm.at[l, k, w], sw[j])

        def wait_writes(l, j):
            for k in range(4):
                pltpu.make_async_copy(tiles[j].at[pl.ds(k * 1024, 1024)],
                                      out_hbm.at[l, k, w], sw[j]).wait()

        for j in range(4):
            fire_gather(j, j)

        @pl.loop(0, _MAXLEN // 4)
        def _(g):
            l0 = 4 * g
            for j in range(4):
                l = l0 + j

                wait_gather(l, j)

                @pl.when(g >= 1)
                def _():
                    wait_writes(l - 4, j)

                @pl.loop(0, _BB, step=8)
                def _(p0):
                    for t in range(8):
                        p = p0 + t
                        r0 = rows[j][p, pl.ds(0, 16)]
                        r1 = rows[j][p, pl.ds(16, 16)]
                        plsc.store_scatter(tiles[j], [sidx0 + p], r0)
                        plsc.store_scatter(tiles[j], [sidx1 + p], r1)

                @pl.when(g < _MAXLEN // 4 - 1)
                def _():
                    fire_gather(l + 4, j)

                fire_writes(l, j)

        for j in range(4):
            wait_writes(_MAXLEN - 4 + j, j)

    return sc_kernel(idx3d, table)


def _tc_reduce_body(raw_ref, avg_ref):
    i = pl.program_id(0)

    @pl.when(i == 0)
    def _():
        avg_ref[...] = jnp.zeros_like(avg_ref)

    avg_ref[...] += jnp.sum(raw_ref[...], axis=0)

    @pl.when(i == pl.num_programs(0) - 1)
    def _():
        avg_ref[...] *= 1.0 / _MAXLEN


def _tc_reduce(raw5):
    lblk = 8
    return pl.pallas_call(
        _tc_reduce_body,
        grid=(_MAXLEN // lblk,),
        in_specs=[
            pl.BlockSpec((lblk, 4, _NW, 8, 128), lambda i: (i, 0, 0, 0, 0))
        ],
        out_specs=pl.BlockSpec((4, _NW, 8, 128), lambda i: (0, 0, 0, 0)),
        out_shape=jax.ShapeDtypeStruct((4, _NW, 8, 128), jnp.float32),
    )(raw5)


def _tc_head_body(xavg_ref, gamma_ref, beta_ref, mean_ref, var_ref, fcw_ref,
                  fcb_ref, bn_ref, final_ref):
    x_avg = xavg_ref[...]
    bn = (x_avg - mean_ref[...]) / jnp.sqrt(var_ref[...] + _BN_EPS) \
        * gamma_ref[...] + beta_ref[...]
    bn_ref[...] = bn
    final_ref[...] = lax.dot_general(
        bn, fcw_ref[...],
        dimension_numbers=(((1,), (1,)), ((), ())),
        preferred_element_type=jnp.float32,
    ) + fcb_ref[...]


def _tc_head(x_avg, bn_gamma, bn_beta, bn_mean, bn_var, fc_w, fc_b):
    f32 = jnp.float32
    return pl.pallas_call(
        _tc_head_body,
        out_shape=[
            jax.ShapeDtypeStruct((_B, _DIM), f32),
            jax.ShapeDtypeStruct((_B, _NCLS), f32),
        ],
    )(
        x_avg,
        bn_gamma.reshape(1, _DIM),
        bn_beta.reshape(1, _DIM),
        bn_mean.reshape(1, _DIM),
        bn_var.reshape(1, _DIM),
        fc_w,
        fc_b.reshape(1, _NCLS),
    )


def kernel(word_idx, table, bn_gamma, bn_beta, bn_mean, bn_var, fc_w, fc_b):
    idx3d = word_idx.reshape(_NW, _BB, _MAXLEN).transpose(0, 2, 1)
    out = _sc_gather_transpose(idx3d, table)
    raw5 = out.reshape(_MAXLEN, 4, _NW, 8, 128)
    raw_feature = raw5.transpose(2, 4, 0, 1, 3).reshape(_B, _MAXLEN, _DIM)
    avg4 = _tc_reduce(raw5)
    x_avg = avg4.transpose(1, 3, 0, 2).reshape(_B, _DIM)
    x_avg_bn, x_final = _tc_head(
        x_avg, bn_gamma, bn_beta, bn_mean, bn_var, fc_w, fc_b
    )
    return (x_final, x_avg_bn, x_avg, raw_feature)
